# baseline (speedup 1.0000x reference)
import functools

import jax
import jax.numpy as jnp
from jax import lax
from jax.experimental import pallas as pl
from jax.experimental.pallas import tpu as pltpu
from jax.experimental.pallas import tpu_sc as plsc

_F = 8
_D = 32
_B = 16384
_C = 64
_PACK = 4


@functools.cache
def _build():
  info = plsc.get_sparse_core_info()
  nc, ns = info.num_cores, info.num_subcores
  nw = nc * ns
  n = _B // nw
  nq = n // _C
  mesh = plsc.VectorSubcoreMesh(core_axis_name="c", subcore_axis_name="s")

  @functools.partial(
      pl.kernel,
      mesh=mesh,
      out_type=jax.ShapeDtypeStruct((_B, _F * _D), jnp.float32),
      scratch_types=[
          pltpu.VMEM((_F, nq, _C), jnp.int32),
          pltpu.VMEM((_F, _C, 128), jnp.float32),
          pltpu.VMEM((_C, _F * _D), jnp.float32),
          pltpu.VMEM((_F, nq, _C), jnp.int32),
          pltpu.SemaphoreType.DMA,
      ],
  )
  def k(j_hbm, rem_hbm, t0, t1, t2, t3, t4, t5, t6, t7, out_hbm,
        j_v, rows_v, big_v, rem_v, sem):
    tables = (t0, t1, t2, t3, t4, t5, t6, t7)
    wid = lax.axis_index("s") * nc + lax.axis_index("c")
    base = wid * n

    pltpu.sync_copy(j_hbm.at[wid], j_v)
    pltpu.sync_copy(rem_hbm.at[wid], rem_v)

    def chunk(q, _):
      handles = [
          pltpu.async_copy(tables[f].at[j_v.at[f, q]], rows_v.at[f], sem)
          for f in range(_F)
      ]
      for h in handles:
        h.wait()

      def body(r16, _):
        rbase = r16 * 16
        for f in range(_F):
          offs = rem_v[f, q, pl.ds(rbase, 16)] * _D
          for kk in range(16):
            r = rbase + kk
            off = offs[kk]
            big_v[r, pl.ds(f * _D, 16)] = rows_v[f, r, pl.ds(off, 16)]
            big_v[r, pl.ds(f * _D + 16, 16)] = (
                rows_v[f, r, pl.ds(off + 16, 16)])
        return 0

      lax.fori_loop(0, _C // 16, body, 0)
      pltpu.sync_copy(big_v, out_hbm.at[pl.ds(base + q * _C, _C), :])
      return 0

    lax.fori_loop(0, nq, chunk, 0)

  return k, nw, nq


def kernel(f0, f1, f2, f3, f4, f5, f6, f7,
           W_f0, W_f1, W_f2, W_f3, W_f4, W_f5, W_f6, W_f7):
  k, nw, nq = _build()
  idx = jnp.stack([f0, f1, f2, f3, f4, f5, f6, f7]).astype(jnp.int32)
  j = (idx >> 2).reshape(_F, nw, nq, _C).transpose(1, 0, 2, 3)
  rem = (idx & 3).reshape(_F, nw, nq, _C).transpose(1, 0, 2, 3)
  packed = [w.reshape(100000 // _PACK, _D * _PACK)
            for w in (W_f0, W_f1, W_f2, W_f3, W_f4, W_f5, W_f6, W_f7)]
  return k(j, rem, *packed)


# speedup vs baseline: 1.0198x; 1.0198x over previous
import functools

import jax
import jax.numpy as jnp
from jax import lax
from jax.experimental import pallas as pl
from jax.experimental.pallas import tpu as pltpu
from jax.experimental.pallas import tpu_sc as plsc

_F = 8
_D = 32
_B = 16384
_C = 64
_PACK = 4


@functools.cache
def _build():
  info = plsc.get_sparse_core_info()
  nc, ns = info.num_cores, info.num_subcores
  nw = nc * ns
  n = _B // nw
  nq = n // _C
  mesh = plsc.VectorSubcoreMesh(core_axis_name="c", subcore_axis_name="s")

  @functools.partial(
      pl.kernel,
      mesh=mesh,
      out_type=jax.ShapeDtypeStruct((_B, _F * _D), jnp.float32),
      scratch_types=[
          pltpu.VMEM((_F, nq, _C), jnp.int32),
          pltpu.VMEM((_F, nq, _C), jnp.int32),
          pltpu.VMEM((2, _C, 128), jnp.float32),
          pltpu.VMEM((_C, _F * _D), jnp.float32),
          pltpu.SemaphoreType.DMA,
          pltpu.SemaphoreType.DMA,
      ],
  )
  def k(j_hbm, rem_hbm, t0, t1, t2, t3, t4, t5, t6, t7, out_hbm,
        j_v, rem_v, rows_v, big_v, sem0, sem1):
    tables = (t0, t1, t2, t3, t4, t5, t6, t7)
    sems = (sem0, sem1)
    wid = lax.axis_index("c") * ns + lax.axis_index("s")
    base = wid * n

    pltpu.sync_copy(j_hbm.at[wid], j_v)
    pltpu.sync_copy(rem_hbm.at[wid], rem_v)

    def start(f, q):
      pltpu.async_copy(tables[f].at[j_v.at[f, q]], rows_v.at[f % 2],
                       sems[f % 2])

    def drain(f, q):
      pltpu.make_async_copy(tables[f].at[j_v.at[f, q]], rows_v.at[f % 2],
                            sems[f % 2]).wait()

    start(0, 0)

    def chunk(q, _):
      for f in range(_F):
        if f + 1 < _F:
          start(f + 1, q)
        else:
          @pl.when(q + 1 < nq)
          def _():
            start(0, q + 1)
        drain(f, q)
        rows = rows_v.at[f % 2]

        def body(r16, _, f=f, rows=rows):
          rbase = r16 * 16
          offs = rem_v[f, q, pl.ds(rbase, 16)] * _D
          for kk in range(16):
            r = rbase + kk
            off = offs[kk]
            big_v[r, pl.ds(f * _D, 16)] = rows[r, pl.ds(off, 16)]
            big_v[r, pl.ds(f * _D + 16, 16)] = rows[r, pl.ds(off + 16, 16)]
          return 0

        lax.fori_loop(0, _C // 16, body, 0)
      pltpu.sync_copy(big_v, out_hbm.at[pl.ds(base + q * _C, _C), :])
      return 0

    lax.fori_loop(0, nq, chunk, 0)

  return k, nw, nq


def kernel(f0, f1, f2, f3, f4, f5, f6, f7,
           W_f0, W_f1, W_f2, W_f3, W_f4, W_f5, W_f6, W_f7):
  k, nw, nq = _build()
  idx = jnp.stack([f0, f1, f2, f3, f4, f5, f6, f7]).astype(jnp.int32)
  j = (idx >> 2).reshape(_F, nw, nq, _C).transpose(1, 0, 2, 3)
  rem = (idx & 3).reshape(_F, nw, nq, _C).transpose(1, 0, 2, 3)
  packed = [w.reshape(100000 // _PACK, _D * _PACK)
            for w in (W_f0, W_f1, W_f2, W_f3, W_f4, W_f5, W_f6, W_f7)]
  return k(j, rem, *packed)


# speedup vs baseline: 1.0441x; 1.0239x over previous
import functools

import jax
import jax.numpy as jnp
from jax import lax
from jax.experimental import pallas as pl
from jax.experimental.pallas import tpu as pltpu
from jax.experimental.pallas import tpu_sc as plsc

_F = 8
_D = 32
_B = 16384
_C = 64
_K = 4
_PACK = 4


@functools.cache
def _build():
  info = plsc.get_sparse_core_info()
  nc, ns = info.num_cores, info.num_subcores
  nw = nc * ns
  n = _B // nw
  nq = n // _C
  mesh = plsc.VectorSubcoreMesh(core_axis_name="c", subcore_axis_name="s")

  @functools.partial(
      pl.kernel,
      mesh=mesh,
      out_type=jax.ShapeDtypeStruct((_B, _F * _D), jnp.float32),
      scratch_types=[
          pltpu.VMEM((_F, nq, _C), jnp.int32),
          pltpu.VMEM((_F, nq, _C), jnp.int32),
          pltpu.VMEM((_K, _C, 128), jnp.float32),
          pltpu.VMEM((_C, _F * _D), jnp.float32),
          pltpu.SemaphoreType.DMA,
          pltpu.SemaphoreType.DMA,
          pltpu.SemaphoreType.DMA,
          pltpu.SemaphoreType.DMA,
      ],
  )
  def k(j_hbm, rem_hbm, t0, t1, t2, t3, t4, t5, t6, t7, out_hbm,
        j_v, rem_v, rows_v, big_v, sem0, sem1, sem2, sem3):
    tables = (t0, t1, t2, t3, t4, t5, t6, t7)
    sems = (sem0, sem1, sem2, sem3)
    wid = lax.axis_index("c") * ns + lax.axis_index("s")
    base = wid * n

    for f in range(_F):
      pltpu.sync_copy(j_hbm.at[f, wid], j_v.at[f])
      pltpu.sync_copy(rem_hbm.at[f, wid], rem_v.at[f])

    def start(f, q):
      pltpu.async_copy(tables[f].at[j_v.at[f, q]], rows_v.at[f % _K],
                       sems[f % _K])

    def drain(f, q):
      pltpu.make_async_copy(tables[f].at[j_v.at[f, q]], rows_v.at[f % _K],
                            sems[f % _K]).wait()

    for f in range(_K):
      start(f, 0)

    def chunk(q, _):
      for f in range(_F):
        drain(f, q)
        rows = rows_v.at[f % _K]

        def body(r16, _, f=f, rows=rows):
          rbase = r16 * 16
          offs = rem_v[f, q, pl.ds(rbase, 16)] * _D
          for kk in range(16):
            r = rbase + kk
            off = offs[kk]
            big_v[r, pl.ds(f * _D, 16)] = rows[r, pl.ds(off, 16)]
            big_v[r, pl.ds(f * _D + 16, 16)] = rows[r, pl.ds(off + 16, 16)]
          return 0

        lax.fori_loop(0, _C // 16, body, 0)
        if f + _K < _F:
          start(f + _K, q)
        else:
          @pl.when(q + 1 < nq)
          def _(f=f):
            start(f + _K - _F, q + 1)
      pltpu.sync_copy(big_v, out_hbm.at[pl.ds(base + q * _C, _C), :])
      return 0

    lax.fori_loop(0, nq, chunk, 0)

  return k, nw, nq


def kernel(f0, f1, f2, f3, f4, f5, f6, f7,
           W_f0, W_f1, W_f2, W_f3, W_f4, W_f5, W_f6, W_f7):
  k, nw, nq = _build()
  idx = jnp.stack([f0, f1, f2, f3, f4, f5, f6, f7]).astype(jnp.int32)
  j = (idx >> 2).reshape(_F, nw, nq, _C)
  rem = (idx & 3).reshape(_F, nw, nq, _C)
  packed = [w.reshape(100000 // _PACK, _D * _PACK)
            for w in (W_f0, W_f1, W_f2, W_f3, W_f4, W_f5, W_f6, W_f7)]
  return k(j, rem, *packed)
